# Initial kernel scaffold; baseline (speedup 1.0000x reference)
import functools

import jax
import jax.numpy as jnp
from jax import lax
from jax.experimental import pallas as pl
from jax.experimental.pallas import tpu as pltpu
from jax.experimental.pallas import tpu_sc as plsc

N = 10000
E = 320000
D = 128
H = 4
NP = 10240
CH = 80
NCHUNK = 10000 // CH
EPT = 10000
RB = 400
GRID = N // RB

_f32 = jnp.float32


def _proj_block(z, albd_ref, arbd_ref, capl_ref, capr_ref, zl_ref, zr_ref, i):
    zl = jnp.dot(z, albd_ref[...], preferred_element_type=_f32)
    zr = jnp.dot(z, arbd_ref[...], preferred_element_type=_f32)
    zl_ref[...] = zl
    zr_ref[...] = zr
    bl = jnp.max(zl, axis=0, keepdims=True)
    br = jnp.max(zr, axis=0, keepdims=True)

    @pl.when(i == 0)
    def _():
        capl_ref[...] = bl
        capr_ref[...] = br

    @pl.when(i > 0)
    def _():
        capl_ref[...] = jnp.maximum(capl_ref[...], bl)
        capr_ref[...] = jnp.maximum(capr_ref[...], br)


def _tc_first_body(x_ref, wcat_ref, albd_ref, arbd_ref,
                   zcat_ref, zl_ref, zr_ref, capl_ref, capr_ref):
    i = pl.program_id(0)
    z = jnp.dot(x_ref[...], wcat_ref[...], preferred_element_type=_f32)
    zcat_ref[...] = z
    _proj_block(z, albd_ref, arbd_ref, capl_ref, capr_ref, zl_ref, zr_ref, i)


def _tc_first(x, wcat, albd, arbd):
    return pl.pallas_call(
        _tc_first_body,
        grid=(GRID,),
        in_specs=[
            pl.BlockSpec((RB, D), lambda i: (i, 0)),
            pl.BlockSpec((D, H * D), lambda i: (0, 0)),
            pl.BlockSpec((H * D, 16), lambda i: (0, 0)),
            pl.BlockSpec((H * D, 16), lambda i: (0, 0)),
        ],
        out_specs=[
            pl.BlockSpec((RB, H * D), lambda i: (i, 0)),
            pl.BlockSpec((RB, 16), lambda i: (i, 0)),
            pl.BlockSpec((RB, 16), lambda i: (i, 0)),
            pl.BlockSpec((1, 16), lambda i: (0, 0)),
            pl.BlockSpec((1, 16), lambda i: (0, 0)),
        ],
        out_shape=[
            jax.ShapeDtypeStruct((N, H * D), _f32),
            jax.ShapeDtypeStruct((N, 16), _f32),
            jax.ShapeDtypeStruct((N, 16), _f32),
            jax.ShapeDtypeStruct((1, 16), _f32),
            jax.ShapeDtypeStruct((1, 16), _f32),
        ],
    )(x, wcat, albd, arbd)


def _elu(x):
    return jnp.where(x > 0.0, x, jnp.expm1(jnp.minimum(x, 0.0)))


def _gru(xn, hprev, wx_ref, wh_ref, bx_ref, bh_ref):
    gx = jnp.dot(xn, wx_ref[...], preferred_element_type=_f32) + bx_ref[...]
    gh = jnp.dot(hprev, wh_ref[...], preferred_element_type=_f32) + bh_ref[...]
    r = jax.nn.sigmoid(gx[:, :D] + gh[:, :D])
    zt = jax.nn.sigmoid(gx[:, D:2 * D] + gh[:, D:2 * D])
    ng = jnp.tanh(gx[:, 2 * D:] + r * gh[:, 2 * D:])
    return (1.0 - zt) * ng + zt * hprev


def _tc_mid_body(a0_ref, a1_ref, h_ref, wx_ref, wh_ref, bx_ref, bh_ref,
                 wcat_ref, albd_ref, arbd_ref,
                 hn_ref, zcat_ref, zl_ref, zr_ref, capl_ref, capr_ref,
                 *, hdiv, use_gru):
    i = pl.program_id(0)
    xn = _elu((a0_ref[...] + a1_ref[...]) * hdiv)
    if use_gru:
        hn = _gru(xn, h_ref[...], wx_ref, wh_ref, bx_ref, bh_ref)
    else:
        hn = xn
    hn_ref[...] = hn
    z = jnp.dot(hn, wcat_ref[...], preferred_element_type=_f32)
    zcat_ref[...] = z
    _proj_block(z, albd_ref, arbd_ref, capl_ref, capr_ref, zl_ref, zr_ref, i)


def _tc_mid(a0, a1, hprev, wx, wh, bx, bh, wcat, albd, arbd, hdiv, use_gru):
    body = functools.partial(_tc_mid_body, hdiv=hdiv, use_gru=use_gru)
    return pl.pallas_call(
        body,
        grid=(GRID,),
        in_specs=[
            pl.BlockSpec((RB, D), lambda i: (i, 0)),
            pl.BlockSpec((RB, D), lambda i: (i, 0)),
            pl.BlockSpec((RB, D), lambda i: (i, 0)),
            pl.BlockSpec((D, 3 * D), lambda i: (0, 0)),
            pl.BlockSpec((D, 3 * D), lambda i: (0, 0)),
            pl.BlockSpec((1, 3 * D), lambda i: (0, 0)),
            pl.BlockSpec((1, 3 * D), lambda i: (0, 0)),
            pl.BlockSpec((D, H * D), lambda i: (0, 0)),
            pl.BlockSpec((H * D, 16), lambda i: (0, 0)),
            pl.BlockSpec((H * D, 16), lambda i: (0, 0)),
        ],
        out_specs=[
            pl.BlockSpec((RB, D), lambda i: (i, 0)),
            pl.BlockSpec((RB, H * D), lambda i: (i, 0)),
            pl.BlockSpec((RB, 16), lambda i: (i, 0)),
            pl.BlockSpec((RB, 16), lambda i: (i, 0)),
            pl.BlockSpec((1, 16), lambda i: (0, 0)),
            pl.BlockSpec((1, 16), lambda i: (0, 0)),
        ],
        out_shape=[
            jax.ShapeDtypeStruct((N, D), _f32),
            jax.ShapeDtypeStruct((N, H * D), _f32),
            jax.ShapeDtypeStruct((N, 16), _f32),
            jax.ShapeDtypeStruct((N, 16), _f32),
            jax.ShapeDtypeStruct((1, 16), _f32),
            jax.ShapeDtypeStruct((1, 16), _f32),
        ],
    )(a0, a1, hprev, wx, wh, bx, bh, wcat, albd, arbd)


def _tc_final_body(a0_ref, a1_ref, h_ref, wx_ref, wh_ref, bx_ref, bh_ref,
                   w5_ref, b5_ref, out_ref):
    xn = _elu(a0_ref[...] + a1_ref[...])
    hn = _gru(xn, h_ref[...], wx_ref, wh_ref, bx_ref, bh_ref)
    out_ref[...] = jax.nn.sigmoid(
        jnp.dot(hn, w5_ref[...], preferred_element_type=_f32) + b5_ref[...])


def _tc_final(a0, a1, hprev, wx, wh, bx, bh, w5, b5):
    return pl.pallas_call(
        _tc_final_body,
        grid=(GRID,),
        in_specs=[
            pl.BlockSpec((RB, D), lambda i: (i, 0)),
            pl.BlockSpec((RB, D), lambda i: (i, 0)),
            pl.BlockSpec((RB, D), lambda i: (i, 0)),
            pl.BlockSpec((D, 3 * D), lambda i: (0, 0)),
            pl.BlockSpec((D, 3 * D), lambda i: (0, 0)),
            pl.BlockSpec((1, 3 * D), lambda i: (0, 0)),
            pl.BlockSpec((1, 3 * D), lambda i: (0, 0)),
            pl.BlockSpec((D, 1), lambda i: (0, 0)),
            pl.BlockSpec((1, 1), lambda i: (0, 0)),
        ],
        out_specs=[pl.BlockSpec((RB, 1), lambda i: (i, 0))],
        out_shape=[jax.ShapeDtypeStruct((N, 1), _f32)],
    )(a0, a1, hprev, wx, wh, bx, bh, w5, b5)


def _sc_body(src_hbm, dst_hbm, zl_hbm, zr_hbm, zcat_hbm, cap_hbm, out_hbm,
             capb, idx_s, idx_d, glb, grb, exb, sgb, zb, vb, zv16,
             sem, s_sh, out_sh):
    c = lax.axis_index("c")
    s = lax.axis_index("s")
    zero16 = jnp.zeros((16,), _f32)
    pltpu.sync_copy(cap_hbm, capb)
    capv = capb[...]

    def _zv(i, carry):
        for j in range(8):
            vb[i, pl.ds(j * 16, 16)] = zero16
        return carry

    lax.fori_loop(0, CH, _zv, 0)

    def _zs(i, carry):
        zv16[i] = zero16
        return carry

    lax.fori_loop(0, 640, _zs, 0)

    row0 = s * 640
    for k in range(8):
        pltpu.sync_copy(vb, out_sh.at[pl.ds(row0 + k * CH, CH)])
    pltpu.sync_copy(zv16, s_sh.at[pl.ds(row0, 640)])
    plsc.subcore_barrier()

    def _b_chunk(i, base):
        eb = base + i * CH
        pltpu.sync_copy(src_hbm.at[pl.ds(eb, CH)], idx_s)
        pltpu.sync_copy(dst_hbm.at[pl.ds(eb, CH)], idx_d)
        pltpu.async_copy(zl_hbm.at[idx_s], glb, sem).wait()
        pltpu.async_copy(zr_hbm.at[idx_d], grb, sem).wait()

        def _e(e, carry):
            ee = glb[e] + grb[e]
            ee = jnp.where(ee >= 0.0, ee, 0.2 * ee)
            exb[e] = jnp.exp(ee - capv)
            return carry

        lax.fori_loop(0, CH, _e, 0, unroll=4)
        pltpu.sync_copy(exb, s_sh.at[idx_d], add=True)
        return base

    lax.fori_loop(0, NCHUNK, _b_chunk, s * EPT)
    lax.fori_loop(0, NCHUNK, _b_chunk, (E // 2) + s * EPT)
    plsc.subcore_barrier()

    def _c_chunk(i, base):
        eb = base + i * CH
        pltpu.sync_copy(src_hbm.at[pl.ds(eb, CH)], idx_s)
        pltpu.sync_copy(dst_hbm.at[pl.ds(eb, CH)], idx_d)
        pltpu.async_copy(zl_hbm.at[idx_s], glb, sem).wait()
        pltpu.async_copy(zr_hbm.at[idx_d], grb, sem).wait()
        pltpu.async_copy(s_sh.at[idx_d], sgb, sem).wait()
        pltpu.async_copy(zcat_hbm.at[idx_s], zb, sem).wait()

        def _e(e, carry):
            ee = glb[e] + grb[e]
            ee = jnp.where(ee >= 0.0, ee, 0.2 * ee)
            ex = jnp.exp(ee - capv)
            av = ex / (sgb[e] + 1e-9)
            acc = [None] * 8
            for h in range(H):
                ah = jnp.broadcast_to(av[h], (16,))
                for j in range(8):
                    zrow = zb[e, pl.ds(h * D + j * 16, 16)]
                    if h == 0:
                        acc[j] = ah * zrow
                    else:
                        acc[j] = acc[j] + ah * zrow
            for j in range(8):
                vb[e, pl.ds(j * 16, 16)] = acc[j]
            return carry

        lax.fori_loop(0, CH, _e, 0)
        pltpu.sync_copy(vb, out_sh.at[idx_d], add=True)
        return base

    lax.fori_loop(0, NCHUNK, _c_chunk, c * (E // 2) + s * EPT)
    plsc.subcore_barrier()
    pltpu.sync_copy(out_sh.at[pl.ds(row0, 640)],
                    out_hbm.at[c, pl.ds(row0, 640)])


def _sc_layer(src, dst, zl16, zr16, zcat, cap16):
    mesh = plsc.VectorSubcoreMesh(core_axis_name="c", subcore_axis_name="s")
    kern = pl.kernel(
        _sc_body,
        out_type=jax.ShapeDtypeStruct((2, NP, D), _f32),
        mesh=mesh,
        scratch_types=[
            pltpu.VMEM((16,), _f32),
            pltpu.VMEM((CH,), jnp.int32),
            pltpu.VMEM((CH,), jnp.int32),
            pltpu.VMEM((CH, 16), _f32),
            pltpu.VMEM((CH, 16), _f32),
            pltpu.VMEM((CH, 16), _f32),
            pltpu.VMEM((CH, 16), _f32),
            pltpu.VMEM((CH, H * D), _f32),
            pltpu.VMEM((CH, D), _f32),
            pltpu.VMEM((640, 16), _f32),
            pltpu.SemaphoreType.DMA,
            pltpu.VMEM_SHARED((NP, 16), _f32),
            pltpu.VMEM_SHARED((NP, D), _f32),
        ],
    )
    return kern(src, dst, zl16, zr16, zcat, cap16)


def _prep(W, al, ar):
    hn = W.shape[0]
    if hn < H:
        W = jnp.concatenate([W, jnp.zeros((H - hn, D, D), _f32)], 0)
        al = jnp.concatenate([al, jnp.zeros((H - hn, D), _f32)], 0)
        ar = jnp.concatenate([ar, jnp.zeros((H - hn, D), _f32)], 0)
    wcat = jnp.transpose(W, (1, 0, 2)).reshape(D, H * D)
    eye = jnp.eye(16, dtype=_f32)[:H]
    albd = (al[:, :, None] * eye[:, None, :]).reshape(H * D, 16)
    arbd = (ar[:, :, None] * eye[:, None, :]).reshape(H * D, 16)
    return wcat, albd, arbd


def kernel(x, edge_index, W1, al1, ar1, W2, al2, ar2, W3, al3, ar3,
           W4, al4, ar4, gru_Wx, gru_Wh, gru_bx, gru_bh, W5, b5):
    src = edge_index[0]
    dst = edge_index[1]
    bx = gru_bx.reshape(1, 3 * D)
    bh = gru_bh.reshape(1, 3 * D)
    b5r = b5.reshape(1, 1)

    wc1, albd1, arbd1 = _prep(W1, al1, ar1)
    wc2, albd2, arbd2 = _prep(W2, al2, ar2)
    wc3, albd3, arbd3 = _prep(W3, al3, ar3)
    wc4, albd4, arbd4 = _prep(W4, al4, ar4)

    zcat, zl, zr, cl, crr = _tc_first(x, wc1, albd1, arbd1)
    outs = _sc_layer(src, dst, zl, zr, zcat, (cl + crr).reshape(16))
    a0, a1 = outs[0, :N], outs[1, :N]

    h1, zcat, zl, zr, cl, crr = _tc_mid(
        a0, a1, x, gru_Wx, gru_Wh, bx, bh, wc2, albd2, arbd2,
        hdiv=1.0 / H, use_gru=False)
    outs = _sc_layer(src, dst, zl, zr, zcat, (cl + crr).reshape(16))
    a0, a1 = outs[0, :N], outs[1, :N]

    h2, zcat, zl, zr, cl, crr = _tc_mid(
        a0, a1, h1, gru_Wx, gru_Wh, bx, bh, wc3, albd3, arbd3,
        hdiv=1.0 / H, use_gru=True)
    outs = _sc_layer(src, dst, zl, zr, zcat, (cl + crr).reshape(16))
    a0, a1 = outs[0, :N], outs[1, :N]

    h3, zcat, zl, zr, cl, crr = _tc_mid(
        a0, a1, h2, gru_Wx, gru_Wh, bx, bh, wc4, albd4, arbd4,
        hdiv=1.0 / H, use_gru=True)
    outs = _sc_layer(src, dst, zl, zr, zcat, (cl + crr).reshape(16))
    a0, a1 = outs[0, :N], outs[1, :N]

    out = _tc_final(a0, a1, h3, gru_Wx, gru_Wh, bx, bh, W5, b5r)
    return out[0]


# speedup vs baseline: 13.7083x; 13.7083x over previous
import functools

import jax
import jax.numpy as jnp
from jax import lax
from jax.experimental import pallas as pl
from jax.experimental.pallas import tpu as pltpu
from jax.experimental.pallas import tpu_sc as plsc

N = 10000
E = 320000
D = 128
H = 4
NP = 10240
CB = 32
SLABC = 25
SLAB4 = SLABC * CB * H
SLABR = SLABC * CB
RB = 400
GRID = N // RB
EHALF = E // 2

_f32 = jnp.float32
_i32 = jnp.int32


def _proj_block(z, albd_ref, arbd_ref, capl_ref, capr_ref, zl_ref, zr_ref, i):
    zl = jnp.dot(z, albd_ref[...], preferred_element_type=_f32)
    zr = jnp.dot(z, arbd_ref[...], preferred_element_type=_f32)
    zl_ref[...] = zl
    zr_ref[...] = zr
    bl = jnp.max(zl, axis=0, keepdims=True)
    br = jnp.max(zr, axis=0, keepdims=True)

    @pl.when(i == 0)
    def _():
        capl_ref[...] = bl
        capr_ref[...] = br

    @pl.when(i > 0)
    def _():
        capl_ref[...] = jnp.maximum(capl_ref[...], bl)
        capr_ref[...] = jnp.maximum(capr_ref[...], br)


def _tc_first_body(x_ref, wcat_ref, albd_ref, arbd_ref,
                   zcat_ref, zl_ref, zr_ref, capl_ref, capr_ref):
    i = pl.program_id(0)
    z = jnp.dot(x_ref[...], wcat_ref[...], preferred_element_type=_f32)
    zcat_ref[...] = z
    _proj_block(z, albd_ref, arbd_ref, capl_ref, capr_ref, zl_ref, zr_ref, i)


def _tc_first(x, wcat, albd, arbd):
    return pl.pallas_call(
        _tc_first_body,
        grid=(GRID,),
        in_specs=[
            pl.BlockSpec((RB, D), lambda i: (i, 0)),
            pl.BlockSpec((D, H * D), lambda i: (0, 0)),
            pl.BlockSpec((H * D, H), lambda i: (0, 0)),
            pl.BlockSpec((H * D, H), lambda i: (0, 0)),
        ],
        out_specs=[
            pl.BlockSpec((RB, H * D), lambda i: (i, 0)),
            pl.BlockSpec((RB, H), lambda i: (i, 0)),
            pl.BlockSpec((RB, H), lambda i: (i, 0)),
            pl.BlockSpec((1, H), lambda i: (0, 0)),
            pl.BlockSpec((1, H), lambda i: (0, 0)),
        ],
        out_shape=[
            jax.ShapeDtypeStruct((N, H * D), _f32),
            jax.ShapeDtypeStruct((NP, H), _f32),
            jax.ShapeDtypeStruct((NP, H), _f32),
            jax.ShapeDtypeStruct((1, H), _f32),
            jax.ShapeDtypeStruct((1, H), _f32),
        ],
    )(x, wcat, albd, arbd)


def _elu(x):
    return jnp.where(x > 0.0, x, jnp.exp(jnp.minimum(x, 0.0)) - 1.0)


def _gru(xn, hprev, wx_ref, wh_ref, bx_ref, bh_ref):
    gx = jnp.dot(xn, wx_ref[...], preferred_element_type=_f32) + bx_ref[...]
    gh = jnp.dot(hprev, wh_ref[...], preferred_element_type=_f32) + bh_ref[...]
    r = jax.nn.sigmoid(gx[:, :D] + gh[:, :D])
    zt = jax.nn.sigmoid(gx[:, D:2 * D] + gh[:, D:2 * D])
    ng = jnp.tanh(gx[:, 2 * D:] + r * gh[:, 2 * D:])
    return (1.0 - zt) * ng + zt * hprev


def _tc_mid_body(a0_ref, a1_ref, h_ref, wx_ref, wh_ref, bx_ref, bh_ref,
                 wcat_ref, albd_ref, arbd_ref,
                 hn_ref, zcat_ref, zl_ref, zr_ref, capl_ref, capr_ref,
                 *, hdiv, use_gru):
    i = pl.program_id(0)
    xn = _elu((a0_ref[...] + a1_ref[...]) * hdiv)
    if use_gru:
        hn = _gru(xn, h_ref[...], wx_ref, wh_ref, bx_ref, bh_ref)
    else:
        hn = xn
    hn_ref[...] = hn
    z = jnp.dot(hn, wcat_ref[...], preferred_element_type=_f32)
    zcat_ref[...] = z
    _proj_block(z, albd_ref, arbd_ref, capl_ref, capr_ref, zl_ref, zr_ref, i)


def _tc_mid(a0, a1, hprev, wx, wh, bx, bh, wcat, albd, arbd, hdiv, use_gru):
    body = functools.partial(_tc_mid_body, hdiv=hdiv, use_gru=use_gru)
    return pl.pallas_call(
        body,
        grid=(GRID,),
        in_specs=[
            pl.BlockSpec((RB, D), lambda i: (i, 0)),
            pl.BlockSpec((RB, D), lambda i: (i, 0)),
            pl.BlockSpec((RB, D), lambda i: (i, 0)),
            pl.BlockSpec((D, 3 * D), lambda i: (0, 0)),
            pl.BlockSpec((D, 3 * D), lambda i: (0, 0)),
            pl.BlockSpec((1, 3 * D), lambda i: (0, 0)),
            pl.BlockSpec((1, 3 * D), lambda i: (0, 0)),
            pl.BlockSpec((D, H * D), lambda i: (0, 0)),
            pl.BlockSpec((H * D, H), lambda i: (0, 0)),
            pl.BlockSpec((H * D, H), lambda i: (0, 0)),
        ],
        out_specs=[
            pl.BlockSpec((RB, D), lambda i: (i, 0)),
            pl.BlockSpec((RB, H * D), lambda i: (i, 0)),
            pl.BlockSpec((RB, H), lambda i: (i, 0)),
            pl.BlockSpec((RB, H), lambda i: (i, 0)),
            pl.BlockSpec((1, H), lambda i: (0, 0)),
            pl.BlockSpec((1, H), lambda i: (0, 0)),
        ],
        out_shape=[
            jax.ShapeDtypeStruct((N, D), _f32),
            jax.ShapeDtypeStruct((N, H * D), _f32),
            jax.ShapeDtypeStruct((NP, H), _f32),
            jax.ShapeDtypeStruct((NP, H), _f32),
            jax.ShapeDtypeStruct((1, H), _f32),
            jax.ShapeDtypeStruct((1, H), _f32),
        ],
    )(a0, a1, hprev, wx, wh, bx, bh, wcat, albd, arbd)


def _tc_final_body(a0_ref, a1_ref, h_ref, wx_ref, wh_ref, bx_ref, bh_ref,
                   w5_ref, b5_ref, out_ref):
    xn = _elu(a0_ref[...] + a1_ref[...])
    hn = _gru(xn, h_ref[...], wx_ref, wh_ref, bx_ref, bh_ref)
    out_ref[...] = jax.nn.sigmoid(
        jnp.dot(hn, w5_ref[...], preferred_element_type=_f32) + b5_ref[...])


def _tc_final(a0, a1, hprev, wx, wh, bx, bh, w5, b5):
    return pl.pallas_call(
        _tc_final_body,
        grid=(GRID,),
        in_specs=[
            pl.BlockSpec((RB, D), lambda i: (i, 0)),
            pl.BlockSpec((RB, D), lambda i: (i, 0)),
            pl.BlockSpec((RB, D), lambda i: (i, 0)),
            pl.BlockSpec((D, 3 * D), lambda i: (0, 0)),
            pl.BlockSpec((D, 3 * D), lambda i: (0, 0)),
            pl.BlockSpec((1, 3 * D), lambda i: (0, 0)),
            pl.BlockSpec((1, 3 * D), lambda i: (0, 0)),
            pl.BlockSpec((D, 1), lambda i: (0, 0)),
            pl.BlockSpec((1, 1), lambda i: (0, 0)),
        ],
        out_specs=[pl.BlockSpec((RB, 1), lambda i: (i, 0))],
        out_shape=[jax.ShapeDtypeStruct((N, 1), _f32)],
    )(a0, a1, hprev, wx, wh, bx, bh, w5, b5)


def _softmax_num(glb, grb, g, capv):
    gl = glb[pl.ds(16 * g, 16)]
    gr = grb[pl.ds(16 * g, 16)]
    ee = gl + gr
    ee = jnp.where(ee >= 0.0, ee, 0.2 * ee)
    return jnp.exp(ee - capv)


def _sc_body(s4_hbm, d4_hbm, sr_hbm, dr_hbm, zl_hbm, zr_hbm, zcat_hbm,
             cap_hbm, out_hbm,
             capb, s4slab, d4slab, srslab, ddslab, idx_w, idx_w32,
             glb, grb, exb, sgb, zb, vb, zflat, sem,
             zl_sh, zr_sh, s_sh, out_sh):
    c = lax.axis_index("c")
    t = lax.axis_index("s")
    zero16 = jnp.zeros((16,), _f32)
    pltpu.sync_copy(cap_hbm, capb)
    capv = capb[...]

    pltpu.sync_copy(zl_hbm.at[pl.ds(t * 2560, 2560)],
                    zl_sh.at[pl.ds(t * 2560, 2560)])
    pltpu.sync_copy(zr_hbm.at[pl.ds(t * 2560, 2560)],
                    zr_sh.at[pl.ds(t * 2560, 2560)])

    def _zv(i, carry):
        for j in range(8):
            vb[i, pl.ds(j * 16, 16)] = zero16
        return carry

    lax.fori_loop(0, CB, _zv, 0)

    def _zf(i, carry):
        zflat[pl.ds(i * 16, 16)] = zero16
        return carry

    lax.fori_loop(0, 160, _zf, 0)

    row0 = t * 640
    for k in range(640 // CB):
        pltpu.sync_copy(vb, out_sh.at[pl.ds(row0 + k * CB, CB)])
    pltpu.sync_copy(zflat, s_sh.at[pl.ds(t * 2560, 2560)])
    plsc.subcore_barrier()

    def _b_slab(m, carry):
        base4 = t * (H * 20000) + m * SLAB4
        pltpu.sync_copy(s4_hbm.at[pl.ds(base4, SLAB4)], s4slab)
        pltpu.sync_copy(d4_hbm.at[pl.ds(base4, SLAB4)], d4slab)

        def _b_chunk(k, carry2):
            off = k * (H * CB)
            pltpu.async_copy(zl_sh.at[s4slab.at[pl.ds(off, H * CB)]],
                             glb, sem).wait()
            pltpu.async_copy(zr_sh.at[d4slab.at[pl.ds(off, H * CB)]],
                             grb, sem).wait()
            for g in range(8):
                exb[pl.ds(16 * g, 16)] = _softmax_num(glb, grb, g, capv)
            for g in range(8):
                idx_w[pl.ds(16 * g, 16)] = d4slab[pl.ds(off + 16 * g, 16)]
            pltpu.sync_copy(exb, s_sh.at[idx_w], add=True)
            return carry2

        lax.fori_loop(0, SLABC, _b_chunk, 0)
        return carry

    lax.fori_loop(0, 25, _b_slab, 0)
    plsc.subcore_barrier()

    nch = 312 + jnp.where(t < 8, 1, 0)
    cb0 = 312 * t + jnp.minimum(t, 8)

    def _c_slab(m, carry):
        base4 = c * (H * EHALF) + cb0 * (H * CB) + m * SLAB4
        baser = c * EHALF + cb0 * CB + m * SLABR
        pltpu.sync_copy(s4_hbm.at[pl.ds(base4, SLAB4)], s4slab)
        pltpu.sync_copy(d4_hbm.at[pl.ds(base4, SLAB4)], d4slab)
        pltpu.sync_copy(sr_hbm.at[pl.ds(baser, SLABR)], srslab)
        pltpu.sync_copy(dr_hbm.at[pl.ds(baser, SLABR)], ddslab)

        def _c_chunk(k, carry2):
            off = k * (H * CB)
            offr = k * CB
            pltpu.async_copy(zl_sh.at[s4slab.at[pl.ds(off, H * CB)]],
                             glb, sem).wait()
            pltpu.async_copy(zr_sh.at[d4slab.at[pl.ds(off, H * CB)]],
                             grb, sem).wait()
            pltpu.async_copy(s_sh.at[d4slab.at[pl.ds(off, H * CB)]],
                             sgb, sem).wait()
            pltpu.async_copy(zcat_hbm.at[srslab.at[pl.ds(offr, CB)]],
                             zb, sem).wait()

            def _g(g, carry3):
                ex = _softmax_num(glb, grb, g, capv)
                av = ex / (sgb[pl.ds(16 * g, 16)] + 1e-9)
                for ke in range(4):
                    e = 4 * g + ke
                    acc = [None] * 8
                    for h in range(H):
                        ah = jnp.broadcast_to(av[4 * ke + h], (16,))
                        for j in range(8):
                            zrow = zb[e, pl.ds(h * D + j * 16, 16)]
                            if h == 0:
                                acc[j] = ah * zrow
                            else:
                                acc[j] = acc[j] + ah * zrow
                    for j in range(8):
                        vb[e, pl.ds(j * 16, 16)] = acc[j]
                return carry3

            lax.fori_loop(0, 8, _g, 0)
            for g in range(2):
                idx_w32[pl.ds(16 * g, 16)] = ddslab[pl.ds(offr + 16 * g, 16)]

            @pl.when(m * SLABC + k < nch)
            def _():
                pltpu.sync_copy(vb, out_sh.at[idx_w32], add=True)

            return carry2

        lax.fori_loop(0, SLABC, _c_chunk, 0)
        return carry

    lax.fori_loop(0, 13, _c_slab, 0)
    plsc.subcore_barrier()
    pltpu.sync_copy(out_sh.at[pl.ds(row0, 640)],
                    out_hbm.at[c, pl.ds(row0, 640)])


def _sc_layer(s4, d4, srp, drp, zlf, zrf, zcat, cap16):
    mesh = plsc.VectorSubcoreMesh(core_axis_name="c", subcore_axis_name="s")
    kern = pl.kernel(
        _sc_body,
        out_type=jax.ShapeDtypeStruct((2, NP, D), _f32),
        mesh=mesh,
        scratch_types=[
            pltpu.VMEM((16,), _f32),
            pltpu.VMEM((SLAB4,), _i32),
            pltpu.VMEM((SLAB4,), _i32),
            pltpu.VMEM((SLABR,), _i32),
            pltpu.VMEM((SLABR,), _i32),
            pltpu.VMEM((H * CB,), _i32),
            pltpu.VMEM((CB,), _i32),
            pltpu.VMEM((H * CB,), _f32),
            pltpu.VMEM((H * CB,), _f32),
            pltpu.VMEM((H * CB,), _f32),
            pltpu.VMEM((H * CB,), _f32),
            pltpu.VMEM((CB, H * D), _f32),
            pltpu.VMEM((CB, D), _f32),
            pltpu.VMEM((2560,), _f32),
            pltpu.SemaphoreType.DMA,
            pltpu.VMEM_SHARED((NP * H,), _f32),
            pltpu.VMEM_SHARED((NP * H,), _f32),
            pltpu.VMEM_SHARED((NP * H,), _f32),
            pltpu.VMEM_SHARED((NP, D), _f32),
        ],
    )
    return kern(s4, d4, srp, drp, zlf, zrf, zcat, cap16)


def _prep(W, al, ar):
    hn = W.shape[0]
    if hn < H:
        W = jnp.concatenate([W, jnp.zeros((H - hn, D, D), _f32)], 0)
        al = jnp.concatenate([al, jnp.zeros((H - hn, D), _f32)], 0)
        ar = jnp.concatenate([ar, jnp.zeros((H - hn, D), _f32)], 0)
    wcat = jnp.transpose(W, (1, 0, 2)).reshape(D, H * D)
    eye = jnp.eye(H, dtype=_f32)
    albd = (al[:, :, None] * eye[:, None, :]).reshape(H * D, H)
    arbd = (ar[:, :, None] * eye[:, None, :]).reshape(H * D, H)
    return wcat, albd, arbd


def kernel(x, edge_index, W1, al1, ar1, W2, al2, ar2, W3, al3, ar3,
           W4, al4, ar4, gru_Wx, gru_Wh, gru_bx, gru_bh, W5, b5):
    src = edge_index[0]
    dst = edge_index[1]
    heads = jnp.arange(H, dtype=_i32)
    s4 = (src[:, None] * H + heads[None, :]).reshape(-1)
    d4 = (dst[:, None] * H + heads[None, :]).reshape(-1)
    s4 = jnp.concatenate([s4, jnp.zeros((SLAB4,), _i32)])
    d4 = jnp.concatenate([d4, jnp.zeros((SLAB4,), _i32)])
    srp = jnp.concatenate([src, jnp.zeros((SLABR,), _i32)])
    drp = jnp.concatenate([dst, jnp.zeros((SLABR,), _i32)])

    bx = gru_bx.reshape(1, 3 * D)
    bh = gru_bh.reshape(1, 3 * D)
    b5r = b5.reshape(1, 1)

    wc1, albd1, arbd1 = _prep(W1, al1, ar1)
    wc2, albd2, arbd2 = _prep(W2, al2, ar2)
    wc3, albd3, arbd3 = _prep(W3, al3, ar3)
    wc4, albd4, arbd4 = _prep(W4, al4, ar4)

    def cap16(cl, crr):
        return jnp.tile((cl + crr).reshape(H), H)

    zcat, zl, zr, cl, crr = _tc_first(x, wc1, albd1, arbd1)
    outs = _sc_layer(s4, d4, srp, drp, zl.reshape(-1), zr.reshape(-1),
                     zcat, cap16(cl, crr))
    a0, a1 = outs[0, :N], outs[1, :N]

    h1, zcat, zl, zr, cl, crr = _tc_mid(
        a0, a1, x, gru_Wx, gru_Wh, bx, bh, wc2, albd2, arbd2,
        hdiv=1.0 / H, use_gru=False)
    outs = _sc_layer(s4, d4, srp, drp, zl.reshape(-1), zr.reshape(-1),
                     zcat, cap16(cl, crr))
    a0, a1 = outs[0, :N], outs[1, :N]

    h2, zcat, zl, zr, cl, crr = _tc_mid(
        a0, a1, h1, gru_Wx, gru_Wh, bx, bh, wc3, albd3, arbd3,
        hdiv=1.0 / H, use_gru=True)
    outs = _sc_layer(s4, d4, srp, drp, zl.reshape(-1), zr.reshape(-1),
                     zcat, cap16(cl, crr))
    a0, a1 = outs[0, :N], outs[1, :N]

    h3, zcat, zl, zr, cl, crr = _tc_mid(
        a0, a1, h2, gru_Wx, gru_Wh, bx, bh, wc4, albd4, arbd4,
        hdiv=1.0 / H, use_gru=True)
    outs = _sc_layer(s4, d4, srp, drp, zl.reshape(-1), zr.reshape(-1),
                     zcat, cap16(cl, crr))
    a0, a1 = outs[0, :N], outs[1, :N]

    out = _tc_final(a0, a1, h3, gru_Wx, gru_Wh, bx, bh, W5, b5r)
    return out[0]


# speedup vs baseline: 15.8661x; 1.1574x over previous
import functools

import jax
import jax.numpy as jnp
from jax import lax
from jax.experimental import pallas as pl
from jax.experimental.pallas import tpu as pltpu
from jax.experimental.pallas import tpu_sc as plsc

N = 10000
E = 320000
D = 128
H = 4
NP = 10240
CB = 32
SLABC = 25
SLAB4 = SLABC * CB * H
SLABR = SLABC * CB
RB = 400
GRID = N // RB
EHALF = E // 2

_f32 = jnp.float32
_i32 = jnp.int32


def _proj_block(z, albd_ref, arbd_ref, capl_ref, capr_ref, zl_ref, zr_ref, i):
    zl = jnp.dot(z, albd_ref[...], preferred_element_type=_f32)
    zr = jnp.dot(z, arbd_ref[...], preferred_element_type=_f32)
    zl_ref[...] = zl
    zr_ref[...] = zr
    bl = jnp.max(zl, axis=0, keepdims=True)
    br = jnp.max(zr, axis=0, keepdims=True)

    @pl.when(i == 0)
    def _():
        capl_ref[...] = bl
        capr_ref[...] = br

    @pl.when(i > 0)
    def _():
        capl_ref[...] = jnp.maximum(capl_ref[...], bl)
        capr_ref[...] = jnp.maximum(capr_ref[...], br)


def _tc_first_body(x_ref, wcat_ref, albd_ref, arbd_ref,
                   zcat_ref, zl_ref, zr_ref, capl_ref, capr_ref):
    i = pl.program_id(0)
    z = jnp.dot(x_ref[...], wcat_ref[...], preferred_element_type=_f32)
    zcat_ref[...] = z
    _proj_block(z, albd_ref, arbd_ref, capl_ref, capr_ref, zl_ref, zr_ref, i)


def _tc_first(x, wcat, albd, arbd):
    return pl.pallas_call(
        _tc_first_body,
        grid=(GRID,),
        in_specs=[
            pl.BlockSpec((RB, D), lambda i: (i, 0)),
            pl.BlockSpec((D, H * D), lambda i: (0, 0)),
            pl.BlockSpec((H * D, H), lambda i: (0, 0)),
            pl.BlockSpec((H * D, H), lambda i: (0, 0)),
        ],
        out_specs=[
            pl.BlockSpec((RB, H * D), lambda i: (i, 0)),
            pl.BlockSpec((RB, H), lambda i: (i, 0)),
            pl.BlockSpec((RB, H), lambda i: (i, 0)),
            pl.BlockSpec((1, H), lambda i: (0, 0)),
            pl.BlockSpec((1, H), lambda i: (0, 0)),
        ],
        out_shape=[
            jax.ShapeDtypeStruct((N, H * D), _f32),
            jax.ShapeDtypeStruct((NP, H), _f32),
            jax.ShapeDtypeStruct((NP, H), _f32),
            jax.ShapeDtypeStruct((1, H), _f32),
            jax.ShapeDtypeStruct((1, H), _f32),
        ],
    )(x, wcat, albd, arbd)


def _elu(x):
    return jnp.where(x > 0.0, x, jnp.exp(jnp.minimum(x, 0.0)) - 1.0)


def _gru(xn, hprev, wx_ref, wh_ref, bx_ref, bh_ref):
    gx = jnp.dot(xn, wx_ref[...], preferred_element_type=_f32) + bx_ref[...]
    gh = jnp.dot(hprev, wh_ref[...], preferred_element_type=_f32) + bh_ref[...]
    r = jax.nn.sigmoid(gx[:, :D] + gh[:, :D])
    zt = jax.nn.sigmoid(gx[:, D:2 * D] + gh[:, D:2 * D])
    ng = jnp.tanh(gx[:, 2 * D:] + r * gh[:, 2 * D:])
    return (1.0 - zt) * ng + zt * hprev


def _tc_mid_body(a0_ref, a1_ref, h_ref, wx_ref, wh_ref, bx_ref, bh_ref,
                 wcat_ref, albd_ref, arbd_ref,
                 hn_ref, zcat_ref, zl_ref, zr_ref, capl_ref, capr_ref,
                 *, hdiv, use_gru):
    i = pl.program_id(0)
    xn = _elu((a0_ref[...] + a1_ref[...]) * hdiv)
    if use_gru:
        hn = _gru(xn, h_ref[...], wx_ref, wh_ref, bx_ref, bh_ref)
    else:
        hn = xn
    hn_ref[...] = hn
    z = jnp.dot(hn, wcat_ref[...], preferred_element_type=_f32)
    zcat_ref[...] = z
    _proj_block(z, albd_ref, arbd_ref, capl_ref, capr_ref, zl_ref, zr_ref, i)


def _tc_mid(a0, a1, hprev, wx, wh, bx, bh, wcat, albd, arbd, hdiv, use_gru):
    body = functools.partial(_tc_mid_body, hdiv=hdiv, use_gru=use_gru)
    return pl.pallas_call(
        body,
        grid=(GRID,),
        in_specs=[
            pl.BlockSpec((RB, D), lambda i: (i, 0)),
            pl.BlockSpec((RB, D), lambda i: (i, 0)),
            pl.BlockSpec((RB, D), lambda i: (i, 0)),
            pl.BlockSpec((D, 3 * D), lambda i: (0, 0)),
            pl.BlockSpec((D, 3 * D), lambda i: (0, 0)),
            pl.BlockSpec((1, 3 * D), lambda i: (0, 0)),
            pl.BlockSpec((1, 3 * D), lambda i: (0, 0)),
            pl.BlockSpec((D, H * D), lambda i: (0, 0)),
            pl.BlockSpec((H * D, H), lambda i: (0, 0)),
            pl.BlockSpec((H * D, H), lambda i: (0, 0)),
        ],
        out_specs=[
            pl.BlockSpec((RB, D), lambda i: (i, 0)),
            pl.BlockSpec((RB, H * D), lambda i: (i, 0)),
            pl.BlockSpec((RB, H), lambda i: (i, 0)),
            pl.BlockSpec((RB, H), lambda i: (i, 0)),
            pl.BlockSpec((1, H), lambda i: (0, 0)),
            pl.BlockSpec((1, H), lambda i: (0, 0)),
        ],
        out_shape=[
            jax.ShapeDtypeStruct((N, D), _f32),
            jax.ShapeDtypeStruct((N, H * D), _f32),
            jax.ShapeDtypeStruct((NP, H), _f32),
            jax.ShapeDtypeStruct((NP, H), _f32),
            jax.ShapeDtypeStruct((1, H), _f32),
            jax.ShapeDtypeStruct((1, H), _f32),
        ],
    )(a0, a1, hprev, wx, wh, bx, bh, wcat, albd, arbd)


def _tc_final_body(a0_ref, a1_ref, h_ref, wx_ref, wh_ref, bx_ref, bh_ref,
                   w5_ref, b5_ref, out_ref):
    xn = _elu(a0_ref[...] + a1_ref[...])
    hn = _gru(xn, h_ref[...], wx_ref, wh_ref, bx_ref, bh_ref)
    out_ref[...] = jax.nn.sigmoid(
        jnp.dot(hn, w5_ref[...], preferred_element_type=_f32) + b5_ref[...])


def _tc_final(a0, a1, hprev, wx, wh, bx, bh, w5, b5):
    return pl.pallas_call(
        _tc_final_body,
        grid=(GRID,),
        in_specs=[
            pl.BlockSpec((RB, D), lambda i: (i, 0)),
            pl.BlockSpec((RB, D), lambda i: (i, 0)),
            pl.BlockSpec((RB, D), lambda i: (i, 0)),
            pl.BlockSpec((D, 3 * D), lambda i: (0, 0)),
            pl.BlockSpec((D, 3 * D), lambda i: (0, 0)),
            pl.BlockSpec((1, 3 * D), lambda i: (0, 0)),
            pl.BlockSpec((1, 3 * D), lambda i: (0, 0)),
            pl.BlockSpec((D, 1), lambda i: (0, 0)),
            pl.BlockSpec((1, 1), lambda i: (0, 0)),
        ],
        out_specs=[pl.BlockSpec((RB, 1), lambda i: (i, 0))],
        out_shape=[jax.ShapeDtypeStruct((N, 1), _f32)],
    )(a0, a1, hprev, wx, wh, bx, bh, w5, b5)


def _softmax_num(glb, grb, g, capv):
    gl = glb[pl.ds(16 * g, 16)]
    gr = grb[pl.ds(16 * g, 16)]
    ee = gl + gr
    ee = jnp.where(ee >= 0.0, ee, 0.2 * ee)
    return jnp.exp(ee - capv)


def _sc_body(s4_hbm, d4_hbm, sr_hbm, dr_hbm, zl_hbm, zr_hbm, zcat_hbm,
             cap_hbm, out_hbm,
             capb, s4slab, d4slab, srslab, ddslab, idx_w, idx_w32,
             glb, grb, exb, sgb, zb, vb, zflat, sem, sem2, sem3, sem4,
             zl_sh, zr_sh, s_sh, out_sh):
    c = lax.axis_index("c")
    t = lax.axis_index("s")
    zero16 = jnp.zeros((16,), _f32)
    pltpu.sync_copy(cap_hbm, capb)
    capv = capb[...]

    pltpu.sync_copy(zl_hbm.at[pl.ds(t * 2560, 2560)],
                    zl_sh.at[pl.ds(t * 2560, 2560)])
    pltpu.sync_copy(zr_hbm.at[pl.ds(t * 2560, 2560)],
                    zr_sh.at[pl.ds(t * 2560, 2560)])

    def _zv(i, carry):
        for j in range(8):
            vb[i, pl.ds(j * 16, 16)] = zero16
        return carry

    lax.fori_loop(0, CB, _zv, 0)

    def _zf(i, carry):
        zflat[pl.ds(i * 16, 16)] = zero16
        return carry

    lax.fori_loop(0, 160, _zf, 0)

    row0 = t * 640
    for k in range(640 // CB):
        pltpu.sync_copy(vb, out_sh.at[pl.ds(row0 + k * CB, CB)])
    pltpu.sync_copy(zflat, s_sh.at[pl.ds(t * 2560, 2560)])
    plsc.subcore_barrier()

    def _b_slab(m, carry):
        base4 = t * (H * 20000) + m * SLAB4
        pltpu.sync_copy(s4_hbm.at[pl.ds(base4, SLAB4)], s4slab)
        pltpu.sync_copy(d4_hbm.at[pl.ds(base4, SLAB4)], d4slab)

        def _b_chunk(k, carry2):
            off = k * (H * CB)
            d1 = pltpu.async_copy(zl_sh.at[s4slab.at[pl.ds(off, H * CB)]],
                                  glb, sem)
            d2 = pltpu.async_copy(zr_sh.at[d4slab.at[pl.ds(off, H * CB)]],
                                  grb, sem2)
            d1.wait()
            d2.wait()
            for g in range(8):
                exb[pl.ds(16 * g, 16)] = _softmax_num(glb, grb, g, capv)
            for g in range(8):
                idx_w[pl.ds(16 * g, 16)] = d4slab[pl.ds(off + 16 * g, 16)]
            pltpu.sync_copy(exb, s_sh.at[idx_w], add=True)
            return carry2

        lax.fori_loop(0, SLABC, _b_chunk, 0)
        return carry

    lax.fori_loop(0, 25, _b_slab, 0)
    plsc.subcore_barrier()

    nch = 312 + jnp.where(t < 8, 1, 0)
    cb0 = 312 * t + jnp.minimum(t, 8)

    def _c_slab(m, carry):
        base4 = c * (H * EHALF) + cb0 * (H * CB) + m * SLAB4
        baser = c * EHALF + cb0 * CB + m * SLABR
        pltpu.sync_copy(s4_hbm.at[pl.ds(base4, SLAB4)], s4slab)
        pltpu.sync_copy(d4_hbm.at[pl.ds(base4, SLAB4)], d4slab)
        pltpu.sync_copy(sr_hbm.at[pl.ds(baser, SLABR)], srslab)
        pltpu.sync_copy(dr_hbm.at[pl.ds(baser, SLABR)], ddslab)

        def _c_chunk(k, carry2):
            off = k * (H * CB)
            offr = k * CB
            dz = pltpu.async_copy(zcat_hbm.at[srslab.at[pl.ds(offr, CB)]],
                                  zb, sem4)
            d1 = pltpu.async_copy(zl_sh.at[s4slab.at[pl.ds(off, H * CB)]],
                                  glb, sem)
            d2 = pltpu.async_copy(zr_sh.at[d4slab.at[pl.ds(off, H * CB)]],
                                  grb, sem2)
            d3 = pltpu.async_copy(s_sh.at[d4slab.at[pl.ds(off, H * CB)]],
                                  sgb, sem3)
            d1.wait()
            d2.wait()
            d3.wait()
            dz.wait()

            def _g(g, carry3):
                ex = _softmax_num(glb, grb, g, capv)
                av = ex / (sgb[pl.ds(16 * g, 16)] + 1e-9)
                for ke in range(4):
                    e = 4 * g + ke
                    acc = [None] * 8
                    for h in range(H):
                        ah = jnp.broadcast_to(av[4 * ke + h], (16,))
                        for j in range(8):
                            zrow = zb[e, pl.ds(h * D + j * 16, 16)]
                            if h == 0:
                                acc[j] = ah * zrow
                            else:
                                acc[j] = acc[j] + ah * zrow
                    for j in range(8):
                        vb[e, pl.ds(j * 16, 16)] = acc[j]
                return carry3

            lax.fori_loop(0, 8, _g, 0)
            for g in range(2):
                idx_w32[pl.ds(16 * g, 16)] = ddslab[pl.ds(offr + 16 * g, 16)]

            @pl.when(m * SLABC + k < nch)
            def _():
                pltpu.sync_copy(vb, out_sh.at[idx_w32], add=True)

            return carry2

        lax.fori_loop(0, SLABC, _c_chunk, 0)
        return carry

    lax.fori_loop(0, 13, _c_slab, 0)
    plsc.subcore_barrier()
    pltpu.sync_copy(out_sh.at[pl.ds(row0, 640)],
                    out_hbm.at[c, pl.ds(row0, 640)])


def _sc_layer(s4, d4, srp, drp, zlf, zrf, zcat, cap16):
    mesh = plsc.VectorSubcoreMesh(core_axis_name="c", subcore_axis_name="s")
    kern = pl.kernel(
        _sc_body,
        out_type=jax.ShapeDtypeStruct((2, NP, D), _f32),
        mesh=mesh,
        scratch_types=[
            pltpu.VMEM((16,), _f32),
            pltpu.VMEM((SLAB4,), _i32),
            pltpu.VMEM((SLAB4,), _i32),
            pltpu.VMEM((SLABR,), _i32),
            pltpu.VMEM((SLABR,), _i32),
            pltpu.VMEM((H * CB,), _i32),
            pltpu.VMEM((CB,), _i32),
            pltpu.VMEM((H * CB,), _f32),
            pltpu.VMEM((H * CB,), _f32),
            pltpu.VMEM((H * CB,), _f32),
            pltpu.VMEM((H * CB,), _f32),
            pltpu.VMEM((CB, H * D), _f32),
            pltpu.VMEM((CB, D), _f32),
            pltpu.VMEM((2560,), _f32),
            pltpu.SemaphoreType.DMA,
            pltpu.SemaphoreType.DMA,
            pltpu.SemaphoreType.DMA,
            pltpu.SemaphoreType.DMA,
            pltpu.VMEM_SHARED((NP * H,), _f32),
            pltpu.VMEM_SHARED((NP * H,), _f32),
            pltpu.VMEM_SHARED((NP * H,), _f32),
            pltpu.VMEM_SHARED((NP, D), _f32),
        ],
    )
    return kern(s4, d4, srp, drp, zlf, zrf, zcat, cap16)


def _prep(W, al, ar):
    hn = W.shape[0]
    if hn < H:
        W = jnp.concatenate([W, jnp.zeros((H - hn, D, D), _f32)], 0)
        al = jnp.concatenate([al, jnp.zeros((H - hn, D), _f32)], 0)
        ar = jnp.concatenate([ar, jnp.zeros((H - hn, D), _f32)], 0)
    wcat = jnp.transpose(W, (1, 0, 2)).reshape(D, H * D)
    eye = jnp.eye(H, dtype=_f32)
    albd = (al[:, :, None] * eye[:, None, :]).reshape(H * D, H)
    arbd = (ar[:, :, None] * eye[:, None, :]).reshape(H * D, H)
    return wcat, albd, arbd


def kernel(x, edge_index, W1, al1, ar1, W2, al2, ar2, W3, al3, ar3,
           W4, al4, ar4, gru_Wx, gru_Wh, gru_bx, gru_bh, W5, b5):
    src = edge_index[0]
    dst = edge_index[1]
    heads = jnp.arange(H, dtype=_i32)
    s4 = (src[:, None] * H + heads[None, :]).reshape(-1)
    d4 = (dst[:, None] * H + heads[None, :]).reshape(-1)
    s4 = jnp.concatenate([s4, jnp.zeros((SLAB4,), _i32)])
    d4 = jnp.concatenate([d4, jnp.zeros((SLAB4,), _i32)])
    srp = jnp.concatenate([src, jnp.zeros((SLABR,), _i32)])
    drp = jnp.concatenate([dst, jnp.zeros((SLABR,), _i32)])

    bx = gru_bx.reshape(1, 3 * D)
    bh = gru_bh.reshape(1, 3 * D)
    b5r = b5.reshape(1, 1)

    wc1, albd1, arbd1 = _prep(W1, al1, ar1)
    wc2, albd2, arbd2 = _prep(W2, al2, ar2)
    wc3, albd3, arbd3 = _prep(W3, al3, ar3)
    wc4, albd4, arbd4 = _prep(W4, al4, ar4)

    def cap16(cl, crr):
        return jnp.tile((cl + crr).reshape(H), H)

    zcat, zl, zr, cl, crr = _tc_first(x, wc1, albd1, arbd1)
    outs = _sc_layer(s4, d4, srp, drp, zl.reshape(-1), zr.reshape(-1),
                     zcat, cap16(cl, crr))
    a0, a1 = outs[0, :N], outs[1, :N]

    h1, zcat, zl, zr, cl, crr = _tc_mid(
        a0, a1, x, gru_Wx, gru_Wh, bx, bh, wc2, albd2, arbd2,
        hdiv=1.0 / H, use_gru=False)
    outs = _sc_layer(s4, d4, srp, drp, zl.reshape(-1), zr.reshape(-1),
                     zcat, cap16(cl, crr))
    a0, a1 = outs[0, :N], outs[1, :N]

    h2, zcat, zl, zr, cl, crr = _tc_mid(
        a0, a1, h1, gru_Wx, gru_Wh, bx, bh, wc3, albd3, arbd3,
        hdiv=1.0 / H, use_gru=True)
    outs = _sc_layer(s4, d4, srp, drp, zl.reshape(-1), zr.reshape(-1),
                     zcat, cap16(cl, crr))
    a0, a1 = outs[0, :N], outs[1, :N]

    h3, zcat, zl, zr, cl, crr = _tc_mid(
        a0, a1, h2, gru_Wx, gru_Wh, bx, bh, wc4, albd4, arbd4,
        hdiv=1.0 / H, use_gru=True)
    outs = _sc_layer(s4, d4, srp, drp, zl.reshape(-1), zr.reshape(-1),
                     zcat, cap16(cl, crr))
    a0, a1 = outs[0, :N], outs[1, :N]

    out = _tc_final(a0, a1, h3, gru_Wx, gru_Wh, bx, bh, W5, b5r)
    return out[0]
